# baseline (speedup 1.0000x reference)
import jax
import jax.numpy as jnp
from jax import lax
from jax.experimental import pallas as pl

_IN_CH = 128
_C = 512
_R = 1.0
_IOU = 0.5
_K = 64
_N = 20000
_NPAD = 20480
_NB = 2048
_CB = 64


def _d2_kernel(c_ref, p_ref, o_ref):
    c = c_ref[...]
    p = p_ref[...]
    cx = c[:, 0:1]
    cy = c[:, 1:2]
    cz = c[:, 2:3]
    px = p[0:1, :]
    py = p[1:2, :]
    pz = p[2:3, :]
    o_ref[...] = ((cx - px) ** 2 + (cy - py) ** 2) + (cz - pz) ** 2


def _head_kernel(x_ref, v_ref, w1_ref, b1_ref, w2_ref, b2_ref, w3_ref,
                 b3_ref, w4_ref, b4_ref, wf_ref, bf_ref, o_ref):
    x = x_ref[...]
    cb, k, f = x.shape
    x2 = x.reshape(cb * k, f)
    h = jnp.dot(x2, w1_ref[...], preferred_element_type=jnp.float32)
    h = jnp.maximum(h + b1_ref[...], 0.0)
    h = jnp.dot(h, w2_ref[...], preferred_element_type=jnp.float32)
    h = jnp.maximum(h + b2_ref[...], 0.0)
    h = h.reshape(cb, k, _IN_CH)
    valid = v_ref[...][:, :, None] > 0.0
    h = jnp.where(valid, h, -jnp.inf)
    g = jnp.max(h, axis=1)
    g = jnp.dot(g, w3_ref[...], preferred_element_type=jnp.float32)
    g = jnp.maximum(g + b3_ref[...], 0.0)
    g = jnp.dot(g, w4_ref[...], preferred_element_type=jnp.float32)
    g = jnp.maximum(g + b4_ref[...], 0.0)
    boxes = jnp.dot(g, wf_ref[...], preferred_element_type=jnp.float32)
    o_ref[...] = jax.nn.sigmoid(boxes + bf_ref[...])


def _fps(points, n_samples):
    N = points.shape[0]
    idxs0 = jnp.zeros((n_samples,), dtype=jnp.int32)
    dists0 = jnp.full((N,), jnp.inf, dtype=jnp.float32)

    def body(i, state):
        idxs, dists = state
        last = points[idxs[i - 1]]
        d = jnp.sum((points - last[None, :]) ** 2, axis=-1)
        dists = jnp.minimum(dists, d)
        idxs = idxs.at[i].set(jnp.argmax(dists).astype(jnp.int32))
        return (idxs, dists)

    idxs, _ = lax.fori_loop(1, n_samples, body, (idxs0, dists0))
    return idxs


def _nms_mask(coords, scores, thr):
    C = coords.shape[0]
    order = jnp.argsort(-scores)
    b = coords[order]
    mins = jnp.minimum(b[:, :3], b[:, 3:])
    maxs = jnp.maximum(b[:, :3], b[:, 3:])
    vol = jnp.prod(maxs - mins, axis=-1)
    ar = jnp.arange(C)

    def body(i, keep):
        lo = jnp.maximum(mins[i], mins)
        hi = jnp.minimum(maxs[i], maxs)
        inter = jnp.prod(jnp.clip(hi - lo, 0.0, None), axis=-1)
        iou = inter / (vol[i] + vol - inter + 1e-9)
        supp = (iou > thr) & (ar > i) & keep[i]
        return keep & (~supp)

    keep_sorted = lax.fori_loop(0, C, body, jnp.ones((C,), dtype=bool))
    keep = jnp.zeros((C,), dtype=bool).at[order].set(keep_sorted)
    return keep


def kernel(vote_points, vote_features, W1, b1, W2, b2, W3, b3, W4, b4, Wf, bf):
    pts = vote_points
    sampled = _fps(lax.stop_gradient(pts), _C)
    centers = pts[sampled]

    pad = jnp.full((_NPAD - _N, 3), 1e8, dtype=jnp.float32)
    pts_t = jnp.concatenate([pts, pad], axis=0).T
    d2 = pl.pallas_call(
        _d2_kernel,
        grid=(_NPAD // _NB,),
        in_specs=[
            pl.BlockSpec((_C, 3), lambda i: (0, 0)),
            pl.BlockSpec((3, _NB), lambda i: (0, i)),
        ],
        out_specs=pl.BlockSpec((_C, _NB), lambda i: (0, i)),
        out_shape=jax.ShapeDtypeStruct((_C, _NPAD), jnp.float32),
    )(centers, pts_t)

    neg_d, nbr = lax.top_k(-d2, _K)
    valid = ((-neg_d) <= _R * _R).astype(jnp.float32)

    grouped_pts = pts[nbr]
    grouped_feat = vote_features[nbr]
    rel = centers[:, None, :] - grouped_pts / _R
    x = jnp.concatenate([rel, grouped_feat], axis=-1)

    row = lambda a: a.reshape(1, -1)
    full = lambda s: pl.BlockSpec(s, lambda i: (0, 0))
    sb = pl.pallas_call(
        _head_kernel,
        grid=(_C // _CB,),
        in_specs=[
            pl.BlockSpec((_CB, _K, 3 + _IN_CH), lambda i: (i, 0, 0)),
            pl.BlockSpec((_CB, _K), lambda i: (i, 0)),
            full((3 + _IN_CH, _IN_CH)), full((1, _IN_CH)),
            full((_IN_CH, _IN_CH)), full((1, _IN_CH)),
            full((_IN_CH, _IN_CH)), full((1, _IN_CH)),
            full((_IN_CH, _IN_CH)), full((1, _IN_CH)),
            full((_IN_CH, 7)), full((1, 7)),
        ],
        out_specs=pl.BlockSpec((_CB, 7), lambda i: (i, 0)),
        out_shape=jax.ShapeDtypeStruct((_C, 7), jnp.float32),
    )(x, valid, W1, row(b1), W2, row(b2), W3, row(b3), W4, row(b4),
      Wf, row(bf))

    box_scores = sb[:, 0]
    box_coords = sb[:, 1:]
    keep = _nms_mask(lax.stop_gradient(box_coords),
                     lax.stop_gradient(box_scores), _IOU)
    final = jnp.concatenate([box_scores[:, None], box_coords], axis=1)
    return final * keep[:, None].astype(jnp.float32)


# speedup vs baseline: 1.0023x; 1.0023x over previous
import jax
import jax.numpy as jnp
from jax import lax
from jax.experimental import pallas as pl

_IN_CH = 128
_C = 512
_R = 1.0
_IOU = 0.5
_K = 64
_N = 20000
_NPAD = 20480
_NB = 2048
_CB = 64


def _d2_kernel(c_ref, p_ref, o_ref):
    c = c_ref[...]
    p = p_ref[...]
    cx = c[:, 0:1]
    cy = c[:, 1:2]
    cz = c[:, 2:3]
    px = p[0:1, :]
    py = p[1:2, :]
    pz = p[2:3, :]
    o_ref[...] = ((cx - px) ** 2 + (cy - py) ** 2) + (cz - pz) ** 2


def _head_kernel(x_ref, v_ref, w1_ref, b1_ref, w2_ref, b2_ref, w3_ref,
                 b3_ref, w4_ref, b4_ref, wf_ref, bf_ref, o_ref):
    x = x_ref[...]
    cb, k, f = x.shape
    x2 = x.reshape(cb * k, f)
    h = jnp.dot(x2, w1_ref[...], preferred_element_type=jnp.float32)
    h = jnp.maximum(h + b1_ref[...], 0.0)
    h = jnp.dot(h, w2_ref[...], preferred_element_type=jnp.float32)
    h = jnp.maximum(h + b2_ref[...], 0.0)
    h = h.reshape(cb, k, _IN_CH)
    valid = v_ref[...][:, :, None] > 0.0
    h = jnp.where(valid, h, -jnp.inf)
    g = jnp.max(h, axis=1)
    g = jnp.dot(g, w3_ref[...], preferred_element_type=jnp.float32)
    g = jnp.maximum(g + b3_ref[...], 0.0)
    g = jnp.dot(g, w4_ref[...], preferred_element_type=jnp.float32)
    g = jnp.maximum(g + b4_ref[...], 0.0)
    boxes = jnp.dot(g, wf_ref[...], preferred_element_type=jnp.float32)
    o_ref[...] = jax.nn.sigmoid(boxes + bf_ref[...])


def _fps(points, n_samples):
    N = points.shape[0]
    idxs0 = jnp.zeros((n_samples,), dtype=jnp.int32)
    dists0 = jnp.full((N,), jnp.inf, dtype=jnp.float32)

    def body(i, state):
        idxs, dists = state
        last = points[idxs[i - 1]]
        d = jnp.sum((points - last[None, :]) ** 2, axis=-1)
        dists = jnp.minimum(dists, d)
        idxs = idxs.at[i].set(jnp.argmax(dists).astype(jnp.int32))
        return (idxs, dists)

    idxs, _ = lax.fori_loop(1, n_samples, body, (idxs0, dists0))
    return idxs


def _nms_mask(coords, scores, thr):
    C = coords.shape[0]
    order = jnp.argsort(-scores)
    b = coords[order]
    mins = jnp.minimum(b[:, :3], b[:, 3:])
    maxs = jnp.maximum(b[:, :3], b[:, 3:])
    vol = jnp.prod(maxs - mins, axis=-1)
    ar = jnp.arange(C)

    def body(i, keep):
        lo = jnp.maximum(mins[i], mins)
        hi = jnp.minimum(maxs[i], maxs)
        inter = jnp.prod(jnp.clip(hi - lo, 0.0, None), axis=-1)
        iou = inter / (vol[i] + vol - inter + 1e-9)
        supp = (iou > thr) & (ar > i) & keep[i]
        return keep & (~supp)

    keep_sorted = lax.fori_loop(0, C, body, jnp.ones((C,), dtype=bool))
    keep = jnp.zeros((C,), dtype=bool).at[order].set(keep_sorted)
    return keep


def kernel(vote_points, vote_features, W1, b1, W2, b2, W3, b3, W4, b4, Wf, bf):
    pts = vote_points
    sampled = _fps(lax.stop_gradient(pts), _C)
    centers = pts[sampled]

    pad = jnp.full((_NPAD - _N, 3), 1e8, dtype=jnp.float32)
    pts_t = jnp.concatenate([pts, pad], axis=0).T
    d2 = pl.pallas_call(
        _d2_kernel,
        grid=(_NPAD // _NB,),
        in_specs=[
            pl.BlockSpec((_C, 3), lambda i: (0, 0)),
            pl.BlockSpec((3, _NB), lambda i: (0, i)),
        ],
        out_specs=pl.BlockSpec((_C, _NB), lambda i: (0, i)),
        out_shape=jax.ShapeDtypeStruct((_C, _NPAD), jnp.float32),
    )(centers, pts_t)

    neg_d, nbr = lax.top_k(-d2[:, :_N], _K)
    valid = ((-neg_d) <= _R * _R).astype(jnp.float32)

    grouped_pts = pts[nbr]
    grouped_feat = vote_features[nbr]
    rel = centers[:, None, :] - grouped_pts / _R
    x = jnp.concatenate([rel, grouped_feat], axis=-1)

    row = lambda a: a.reshape(1, -1)
    full = lambda s: pl.BlockSpec(s, lambda i: (0, 0))
    sb = pl.pallas_call(
        _head_kernel,
        grid=(_C // _CB,),
        in_specs=[
            pl.BlockSpec((_CB, _K, 3 + _IN_CH), lambda i: (i, 0, 0)),
            pl.BlockSpec((_CB, _K), lambda i: (i, 0)),
            full((3 + _IN_CH, _IN_CH)), full((1, _IN_CH)),
            full((_IN_CH, _IN_CH)), full((1, _IN_CH)),
            full((_IN_CH, _IN_CH)), full((1, _IN_CH)),
            full((_IN_CH, _IN_CH)), full((1, _IN_CH)),
            full((_IN_CH, 7)), full((1, 7)),
        ],
        out_specs=pl.BlockSpec((_CB, 7), lambda i: (i, 0)),
        out_shape=jax.ShapeDtypeStruct((_C, 7), jnp.float32),
    )(x, valid, W1, row(b1), W2, row(b2), W3, row(b3), W4, row(b4),
      Wf, row(bf))

    box_scores = sb[:, 0]
    box_coords = sb[:, 1:]
    keep = _nms_mask(lax.stop_gradient(box_coords),
                     lax.stop_gradient(box_scores), _IOU)
    final = jnp.concatenate([box_scores[:, None], box_coords], axis=1)
    return final * keep[:, None].astype(jnp.float32)
